# baseline (speedup 1.0000x reference)
import functools
import math

import jax
import jax.numpy as jnp
from jax.experimental import pallas as pl
from jax.experimental.pallas import tpu as pltpu

H = 16
F32 = jnp.float32
BF16 = jnp.bfloat16


def _proj_body(k_ref, v_ref, wk_ref, bk_ref, wv_ref, bv_ref, kp_ref, vp_ref):
    kp_ref[...] = (jax.lax.dot_general(
        k_ref[...], wk_ref[...], (((1,), (1,)), ((), ())),
        preferred_element_type=F32) + bk_ref[...]).astype(BF16)
    vp_ref[...] = (jax.lax.dot_general(
        v_ref[...], wv_ref[...], (((1,), (1,)), ((), ())),
        preferred_element_type=F32) + bv_ref[...]).astype(BF16)


def _main_body(scale, nb, R, n, q_ref, b_ref, wq_ref, bq_ref, kp_ref, vp_ref,
               wo_ref, bo_ref, out_ref, ex_scr, den_scr, w_scr):
    i = pl.program_id(0)
    b = b_ref[0, 0, :]
    sp = kp_ref.shape[0]
    seg = jax.lax.broadcasted_iota(jnp.int32, (b.shape[0], sp), 1)
    onehot = (b[:, None] == seg).astype(BF16)

    @pl.when(i < nb)
    def _phase0():
        qp = jax.lax.dot_general(
            q_ref[...], wq_ref[...], (((1,), (1,)), ((), ())),
            preferred_element_type=F32) + bq_ref[...]
        kx = jnp.dot(onehot, kp_ref[...], preferred_element_type=F32)
        row = i * R + jax.lax.broadcasted_iota(jnp.int32, (R, 1), 0)
        ex = jnp.where(row < n, jnp.exp(qp * kx * scale), 0.0).astype(BF16)
        ex_scr[pl.ds(i * R, R), :] = ex
        partial = jax.lax.dot_general(
            onehot, ex, (((0,), (0,)), ((), ())),
            preferred_element_type=F32)

        @pl.when(i == 0)
        def _init():
            den_scr[...] = partial

        @pl.when(i > 0)
        def _acc():
            den_scr[...] += partial

    @pl.when(i == nb)
    def _boundary():
        den = den_scr[...]
        w_scr[...] = jnp.where(
            den > 0.0, vp_ref[...].astype(F32) / den, 0.0).astype(BF16)

    @pl.when(i >= nb)
    def _phase1():
        j = i - nb
        wx = jnp.dot(onehot, w_scr[...], preferred_element_type=F32)
        y = ex_scr[pl.ds(j * R, R), :].astype(F32) * wx
        out_ref[...] = jax.lax.dot_general(
            y, wo_ref[...], (((1,), (1,)), ((), ())),
            preferred_element_type=F32) + bo_ref[...]


def kernel(q, k, v, batch, Wq, bq, Wk, bk, Wv, bv, Wo, bo):
    n, dm = q.shape
    s = k.shape[0]
    d = dm // H
    scale = 1.0 / math.sqrt(float(d))

    R = 256
    nb = -(-n // R)
    npad = nb * R
    sp = -(-(s + 1) // 128) * 128

    bz = jnp.pad(batch.astype(jnp.int32), (0, npad - n),
                 constant_values=sp - 1)
    b3 = bz.reshape(nb, 1, R)
    kz = jnp.pad(k, ((0, sp - s), (0, 0)))
    vz = jnp.pad(v, ((0, sp - s), (0, 0)))
    bq2, bk2, bv2, bo2 = (x.reshape(1, dm) for x in (bq, bk, bv, bo))

    full = lambda *shape: pl.BlockSpec(shape, lambda i: (0,) * len(shape))

    kp, vp = pl.pallas_call(
        _proj_body,
        grid=(1,),
        in_specs=[full(sp, dm), full(sp, dm), full(dm, dm), full(1, dm),
                  full(dm, dm), full(1, dm)],
        out_specs=[full(sp, dm), full(sp, dm)],
        out_shape=[jax.ShapeDtypeStruct((sp, dm), BF16),
                   jax.ShapeDtypeStruct((sp, dm), BF16)],
    )(kz, vz, Wk, bk2, Wv, bv2)

    out = pl.pallas_call(
        functools.partial(_main_body, scale, nb, R, n),
        grid=(2 * nb,),
        in_specs=[
            pl.BlockSpec((R, dm), lambda i: (jnp.where(i < nb, i, 0), 0)),
            pl.BlockSpec((1, 1, R),
                         lambda i: (jnp.where(i < nb, i, i - nb), 0, 0)),
            full(dm, dm), full(1, dm), full(sp, dm), full(sp, dm),
            full(dm, dm), full(1, dm),
        ],
        out_specs=pl.BlockSpec(
            (R, dm), lambda i: (jnp.where(i < nb, 0, i - nb), 0)),
        out_shape=jax.ShapeDtypeStruct((n, dm), F32),
        scratch_shapes=[
            pltpu.VMEM((npad, dm), BF16),
            pltpu.VMEM((sp, dm), F32),
            pltpu.VMEM((sp, dm), BF16),
        ],
        compiler_params=pltpu.CompilerParams(
            dimension_semantics=("arbitrary",),
            vmem_limit_bytes=120 * 1024 * 1024),
    )(q, b3, Wq, bq2, kp, vp, Wo, bo2)

    return out


# speedup vs baseline: 1.3368x; 1.3368x over previous
import functools
import math

import jax
import jax.numpy as jnp
from jax.experimental import pallas as pl
from jax.experimental.pallas import tpu as pltpu

H = 16
F32 = jnp.float32
BF16 = jnp.bfloat16


def _proj_body(k_ref, v_ref, wk_ref, bk_ref, wv_ref, bv_ref, kp_ref, vp_ref):
    kp_ref[...] = (jax.lax.dot_general(
        k_ref[...], wk_ref[...], (((1,), (1,)), ((), ())),
        preferred_element_type=F32) + bk_ref[...]).astype(BF16)
    vp_ref[...] = (jax.lax.dot_general(
        v_ref[...], wv_ref[...], (((1,), (1,)), ((), ())),
        preferred_element_type=F32) + bv_ref[...]).astype(BF16)


def _main_body(scale, nb, R, n, q_ref, b_ref, wq_ref, bq_ref, kp_ref, vp_ref,
               wo_ref, bo_ref, out_ref, ex_scr, den_scr, w_scr):
    i = pl.program_id(0)
    b = b_ref[0, 0, :]
    sp = kp_ref.shape[0]
    seg = jax.lax.broadcasted_iota(jnp.int32, (b.shape[0], sp), 1)
    onehot = (b[:, None] == seg).astype(BF16)

    @pl.when(i < nb)
    def _phase0():
        qp = jax.lax.dot_general(
            q_ref[...].astype(BF16), wq_ref[...], (((1,), (1,)), ((), ())),
            preferred_element_type=F32) + bq_ref[...]
        kx = jnp.dot(onehot, kp_ref[...], preferred_element_type=F32)
        row = i * R + jax.lax.broadcasted_iota(jnp.int32, (R, 1), 0)
        ex = jnp.where(row < n, jnp.exp(qp * kx * scale), 0.0).astype(BF16)
        ex_scr[pl.ds(i * R, R), :] = ex
        partial = jax.lax.dot_general(
            onehot, ex, (((0,), (0,)), ((), ())),
            preferred_element_type=F32)

        @pl.when(i == 0)
        def _init():
            den_scr[...] = partial

        @pl.when(i > 0)
        def _acc():
            den_scr[...] += partial

    @pl.when(i == nb)
    def _boundary():
        den = den_scr[...]
        w_scr[...] = jnp.where(
            den > 0.0, vp_ref[...].astype(F32) / den, 0.0).astype(BF16)

    @pl.when(i >= nb)
    def _phase1():
        j = i - nb
        wx = jnp.dot(onehot, w_scr[...], preferred_element_type=F32)
        y = (ex_scr[pl.ds(j * R, R), :].astype(F32) * wx).astype(BF16)
        out_ref[...] = jax.lax.dot_general(
            y, wo_ref[...], (((1,), (1,)), ((), ())),
            preferred_element_type=F32) + bo_ref[...]


def kernel(q, k, v, batch, Wq, bq, Wk, bk, Wv, bv, Wo, bo):
    n, dm = q.shape
    s = k.shape[0]
    d = dm // H
    scale = 1.0 / math.sqrt(float(d))

    R = 512
    nb = -(-n // R)
    npad = nb * R
    sp = -(-(s + 1) // 128) * 128

    bz = jnp.pad(batch.astype(jnp.int32), (0, npad - n),
                 constant_values=sp - 1)
    b3 = bz.reshape(nb, 1, R)
    kz = jnp.pad(k, ((0, sp - s), (0, 0)))
    vz = jnp.pad(v, ((0, sp - s), (0, 0)))
    bq2, bk2, bv2, bo2 = (x.reshape(1, dm) for x in (bq, bk, bv, bo))

    full = lambda *shape: pl.BlockSpec(shape, lambda i: (0,) * len(shape))

    kp, vp = pl.pallas_call(
        _proj_body,
        grid=(1,),
        in_specs=[full(sp, dm), full(sp, dm), full(dm, dm), full(1, dm),
                  full(dm, dm), full(1, dm)],
        out_specs=[full(sp, dm), full(sp, dm)],
        out_shape=[jax.ShapeDtypeStruct((sp, dm), BF16),
                   jax.ShapeDtypeStruct((sp, dm), BF16)],
    )(kz, vz, Wk, bk2, Wv, bv2)

    out = pl.pallas_call(
        functools.partial(_main_body, scale, nb, R, n),
        grid=(2 * nb,),
        in_specs=[
            pl.BlockSpec((R, dm), lambda i: (jnp.where(i < nb, i, 0), 0)),
            pl.BlockSpec((1, 1, R),
                         lambda i: (jnp.where(i < nb, i, i - nb), 0, 0)),
            full(dm, dm), full(1, dm), full(sp, dm), full(sp, dm),
            full(dm, dm), full(1, dm),
        ],
        out_specs=pl.BlockSpec(
            (R, dm), lambda i: (jnp.where(i < nb, 0, i - nb), 0)),
        out_shape=jax.ShapeDtypeStruct((n, dm), F32),
        scratch_shapes=[
            pltpu.VMEM((npad, dm), BF16),
            pltpu.VMEM((sp, dm), F32),
            pltpu.VMEM((sp, dm), BF16),
        ],
        compiler_params=pltpu.CompilerParams(
            dimension_semantics=("arbitrary",),
            vmem_limit_bytes=120 * 1024 * 1024),
    )(q, b3, Wq.astype(BF16), bq2, kp, vp, Wo.astype(BF16), bo2)

    return out


# speedup vs baseline: 1.8713x; 1.3998x over previous
import functools
import math

import jax
import jax.numpy as jnp
from jax.experimental import pallas as pl
from jax.experimental.pallas import tpu as pltpu

H = 16
W = 128
F32 = jnp.float32
BF16 = jnp.bfloat16


def _proj_body(k_ref, v_ref, wk_ref, bk_ref, wv_ref, bv_ref, kp_ref, vp_ref):
    kp_ref[...] = (jax.lax.dot_general(
        k_ref[...], wk_ref[...], (((1,), (1,)), ((), ())),
        preferred_element_type=F32) + bk_ref[...]).astype(BF16)
    vp_ref[...] = (jax.lax.dot_general(
        v_ref[...], wv_ref[...], (((1,), (1,)), ((), ())),
        preferred_element_type=F32) + bv_ref[...]).astype(BF16)


def _window(b, lo_ref, hi_ref, i, sp):
    lo8 = pl.multiple_of(
        jnp.minimum((lo_ref[i] // 8) * 8, sp - W), 8)
    fits = hi_ref[i] - lo8 < W
    seg_w = lo8 + jax.lax.broadcasted_iota(jnp.int32, (b.shape[0], W), 1)
    oh_w = (b[:, None] == seg_w).astype(BF16)
    return lo8, fits, oh_w


def _pass1_body(scale, R, n, lo_ref, hi_ref, q_ref, b_ref, wq_ref, bq_ref,
                kp_ref, ex_ref, den_ref):
    i = pl.program_id(0)
    qp = jax.lax.dot_general(
        q_ref[...].astype(BF16), wq_ref[...], (((1,), (1,)), ((), ())),
        preferred_element_type=F32) + bq_ref[...]
    b = b_ref[0, 0, :]
    sp = kp_ref.shape[0]
    row = i * R + jax.lax.broadcasted_iota(jnp.int32, (R, 1), 0)
    lo8, fits, oh_w = _window(b, lo_ref, hi_ref, i, sp)

    @pl.when(i == 0)
    def _init():
        den_ref[...] = jnp.zeros_like(den_ref)

    @pl.when(fits)
    def _fast():
        kx = jnp.dot(oh_w, kp_ref[pl.ds(lo8, W), :],
                     preferred_element_type=F32)
        ex = jnp.where(row < n, jnp.exp(qp * kx * scale), 0.0).astype(BF16)
        ex_ref[...] = ex
        den_ref[pl.ds(lo8, W), :] += jax.lax.dot_general(
            oh_w, ex, (((0,), (0,)), ((), ())), preferred_element_type=F32)

    @pl.when(jnp.logical_not(fits))
    def _slow():
        seg = jax.lax.broadcasted_iota(jnp.int32, (b.shape[0], sp), 1)
        oh = (b[:, None] == seg).astype(BF16)
        kx = jnp.dot(oh, kp_ref[...], preferred_element_type=F32)
        ex = jnp.where(row < n, jnp.exp(qp * kx * scale), 0.0).astype(BF16)
        ex_ref[...] = ex
        den_ref[...] += jax.lax.dot_general(
            oh, ex, (((0,), (0,)), ((), ())), preferred_element_type=F32)


def _pass2_body(lo_ref, hi_ref, ex_ref, b_ref, vp_ref, den_ref, wo_ref,
                bo_ref, out_ref):
    i = pl.program_id(0)
    b = b_ref[0, 0, :]
    sp = vp_ref.shape[0]
    lo8, fits, oh_w = _window(b, lo_ref, hi_ref, i, sp)

    def emit(wx):
        y = (ex_ref[...].astype(F32) * wx).astype(BF16)
        out_ref[...] = jax.lax.dot_general(
            y, wo_ref[...], (((1,), (1,)), ((), ())),
            preferred_element_type=F32) + bo_ref[...]

    @pl.when(fits)
    def _fast():
        den = den_ref[pl.ds(lo8, W), :]
        w = jnp.where(den > 0.0,
                      vp_ref[pl.ds(lo8, W), :].astype(F32) / den,
                      0.0).astype(BF16)
        emit(jnp.dot(oh_w, w, preferred_element_type=F32))

    @pl.when(jnp.logical_not(fits))
    def _slow():
        den = den_ref[...]
        w = jnp.where(den > 0.0, vp_ref[...].astype(F32) / den,
                      0.0).astype(BF16)
        seg = jax.lax.broadcasted_iota(jnp.int32, (b.shape[0], sp), 1)
        oh = (b[:, None] == seg).astype(BF16)
        emit(jnp.dot(oh, w, preferred_element_type=F32))


def kernel(q, k, v, batch, Wq, bq, Wk, bk, Wv, bv, Wo, bo):
    n, dm = q.shape
    s = k.shape[0]
    d = dm // H
    scale = 1.0 / math.sqrt(float(d))

    R = 512
    nb = -(-n // R)
    npad = nb * R
    sp = -(-s // 128) * 128

    bi = batch.astype(jnp.int32)
    bz = jnp.pad(bi, (0, npad - n), mode="edge")
    b3 = bz.reshape(nb, 1, R)
    b2 = bz.reshape(nb, R)
    lo = b2[:, 0]
    hi = b2[:, R - 1]
    kz = jnp.pad(k, ((0, sp - s), (0, 0)))
    vz = jnp.pad(v, ((0, sp - s), (0, 0)))
    bq2, bk2, bv2, bo2 = (x.reshape(1, dm) for x in (bq, bk, bv, bo))

    full = lambda *shape: pl.BlockSpec(shape, lambda i: (0,) * len(shape))
    smem = pl.BlockSpec(memory_space=pltpu.SMEM)

    kp, vp = pl.pallas_call(
        _proj_body,
        grid=(1,),
        in_specs=[full(sp, dm), full(sp, dm), full(dm, dm), full(1, dm),
                  full(dm, dm), full(1, dm)],
        out_specs=[full(sp, dm), full(sp, dm)],
        out_shape=[jax.ShapeDtypeStruct((sp, dm), BF16),
                   jax.ShapeDtypeStruct((sp, dm), BF16)],
    )(kz, vz, Wk, bk2, Wv, bv2)

    ex, den = pl.pallas_call(
        functools.partial(_pass1_body, scale, R, n),
        grid=(nb,),
        in_specs=[
            smem, smem,
            pl.BlockSpec((R, dm), lambda i: (i, 0)),
            pl.BlockSpec((1, 1, R), lambda i: (i, 0, 0)),
            full(dm, dm), full(1, dm), full(sp, dm),
        ],
        out_specs=[pl.BlockSpec((R, dm), lambda i: (i, 0)), full(sp, dm)],
        out_shape=[jax.ShapeDtypeStruct((n, dm), BF16),
                   jax.ShapeDtypeStruct((sp, dm), F32)],
        compiler_params=pltpu.CompilerParams(
            dimension_semantics=("arbitrary",)),
    )(lo, hi, q, b3, Wq.astype(BF16), bq2, kp)

    out = pl.pallas_call(
        functools.partial(_pass2_body),
        grid=(nb,),
        in_specs=[
            smem, smem,
            pl.BlockSpec((R, dm), lambda i: (i, 0)),
            pl.BlockSpec((1, 1, R), lambda i: (i, 0, 0)),
            full(sp, dm), full(sp, dm), full(dm, dm), full(1, dm),
        ],
        out_specs=pl.BlockSpec((R, dm), lambda i: (i, 0)),
        out_shape=jax.ShapeDtypeStruct((n, dm), F32),
        compiler_params=pltpu.CompilerParams(
            dimension_semantics=("arbitrary",)),
    )(lo, hi, ex, b3, vp, den, Wo.astype(BF16), bo2)

    return out
